# Initial kernel scaffold; baseline (speedup 1.0000x reference)
import jax
import jax.numpy as jnp
from jax.experimental import pallas as pl


def kernel(x, pe):
    raise NotImplementedError("write your pallas kernel here")


# speedup vs baseline: 7.4545x; 7.4545x over previous
import functools

import jax
import jax.numpy as jnp
from jax import lax
from jax.experimental import pallas as pl
from jax.experimental.pallas import tpu as pltpu
from jax.experimental.pallas import tpu_sc as plsc

D_MODEL = 128
NUM_CORES = 2
NUM_SUBCORES = 16
NW = NUM_CORES * NUM_SUBCORES
CHUNK = 128


@functools.partial(jax.jit, static_argnames=())
def _gather_flat(x_r, pe):
    n_chunks = x_r.shape[1]
    d = pe.shape[1]
    mesh = plsc.VectorSubcoreMesh(
        core_axis_name="c",
        subcore_axis_name="s",
        num_cores=NUM_CORES,
        num_subcores=NUM_SUBCORES,
    )

    @functools.partial(
        pl.kernel,
        mesh=mesh,
        out_type=jax.ShapeDtypeStruct((NW, n_chunks, CHUNK, d), jnp.float32),
        scratch_types=[
            pltpu.VMEM((n_chunks, CHUNK), jnp.int32),
            pltpu.VMEM((CHUNK, d), jnp.float32),
            pltpu.VMEM((CHUNK, d), jnp.float32),
            pltpu.SemaphoreType.DMA,
            pltpu.SemaphoreType.DMA,
        ],
    )
    def k(x_hbm, pe_hbm, out_hbm, idx_v, buf0, buf1, gsem0, gsem1):
        wid = lax.axis_index("s") * NUM_CORES + lax.axis_index("c")
        pltpu.sync_copy(x_hbm.at[wid], idx_v)
        pltpu.async_copy(pe_hbm.at[idx_v.at[0]], buf0, gsem0)

        @pl.loop(0, n_chunks, step=2)
        def body(j):
            pltpu.async_copy(pe_hbm.at[idx_v.at[j + 1]], buf1, gsem1)
            pltpu.make_async_copy(pe_hbm.at[idx_v.at[j]], buf0, gsem0).wait()
            pltpu.sync_copy(buf0, out_hbm.at[wid, j])

            @pl.when(j + 2 < n_chunks)
            def _():
                pltpu.async_copy(pe_hbm.at[idx_v.at[j + 2]], buf0, gsem0)

            pltpu.make_async_copy(pe_hbm.at[idx_v.at[j + 1]], buf1, gsem1).wait()
            pltpu.sync_copy(buf1, out_hbm.at[wid, j + 1])

    return k(x_r, pe)


def kernel(x, pe):
    b, s = x.shape
    total = b * s
    assert total % (NW * CHUNK) == 0
    n_chunks = total // (NW * CHUNK)
    x_r = x.reshape(NW, n_chunks, CHUNK)
    out = _gather_flat(x_r, pe)
    return out.reshape(b, s, pe.shape[1])
